# baseline (speedup 1.0000x reference)
import jax
import jax.numpy as jnp
from jax.experimental import pallas as pl
from jax.experimental.pallas import tpu as pltpu

D_MODEL = 4096
NUM_PATHS = 512
K = 51
T_BLK = 512
SPLIT = 4
V = 64


def _cex(ks, ix, i, j, flip):
    a, b = ks[i], ks[j]
    ia, ib = ix[i], ix[j]
    g = a < b
    if not flip:
        ks[i], ks[j] = jnp.where(g, b, a), jnp.where(g, a, b)
        ix[i], ix[j] = jnp.where(g, ib, ia), jnp.where(g, ia, ib)
    else:
        ks[i], ks[j] = jnp.where(g, a, b), jnp.where(g, b, a)
        ix[i], ix[j] = jnp.where(g, ia, ib), jnp.where(g, ib, ia)


def _sort64_desc(ks, ix):
    def merge(lo, hi, r):
        step = r * 2
        if step < hi - lo:
            merge(lo, hi, step)
            merge(lo + r, hi, step)
            for i in range(lo + r, hi - r, step):
                _cex(ks, ix, i, i + r, flip=False)
        else:
            _cex(ks, ix, lo, lo + r, flip=False)

    def msort(lo, hi):
        if hi - lo > 1:
            mid = (lo + hi) // 2
            msort(lo, mid)
            msort(mid, hi)
            merge(lo, hi, 1)

    msort(0, V)


def _merge64_desc(ks, ix):
    s = V // 2
    while s >= 1:
        for i in range(V):
            if not i & s:
                _cex(ks, ix, i, i | s, flip=False)
        s //= 2


def _subrot(arr, d):
    return pltpu.roll(arr, 8 - d, axis=0)


def _gate_kernel(x_ref, w_ref, idx_ref, scores_ref):
    w = w_ref[...]
    ts = T_BLK // SPLIT
    logits_parts = []
    for p in range(SPLIT):
        xp = x_ref[p * ts:(p + 1) * ts, :]
        logits_parts.append(jax.lax.dot_general(
            w, xp, (((1,), (1,)), ((), ())),
            preferred_element_type=jnp.float32,
        ))
    for p in range(SPLIT):
        _topk_softmax(logits_parts[p], idx_ref, scores_ref, p * ts)


def _topk_softmax(logits, idx_ref, scores_ref, col0):
    t = logits.shape[-1]
    ks = [logits[8 * v: 8 * v + 8, :] for v in range(V)]
    sub = jax.lax.broadcasted_iota(jnp.int32, (8, t), 0)
    ix = [sub + 8 * v for v in range(V)]

    _sort64_desc(ks, ix)

    for d in (1, 2, 4):
        pks = [_subrot(ks[V - 1 - v], d) for v in range(V)]
        pix = [_subrot(ix[V - 1 - v], d) for v in range(V)]
        for v in range(V):
            g = pks[v] > ks[v]
            ks[v] = jnp.where(g, pks[v], ks[v])
            ix[v] = jnp.where(g, pix[v], ix[v])
        _merge64_desc(ks, ix)

    kv = jnp.concatenate([ks[r][0:1, :] for r in range(V)], axis=0)
    iv = jnp.concatenate([ix[r][0:1, :] for r in range(V)], axis=0)

    rank = jax.lax.broadcasted_iota(jnp.int32, (V, t), 0)
    e = jnp.where(rank < K, jnp.exp(kv - kv[0:1, :]), 0.0)
    ssum = jnp.sum(e, axis=0, keepdims=True)
    sc = e / ssum

    idx_ref[:, col0:col0 + t] = iv
    scores_ref[:, col0:col0 + t] = sc


@jax.jit
def kernel(x, W):
    n_tokens = x.shape[0]
    nblk = n_tokens // T_BLK
    idx_t, scores_t = pl.pallas_call(
        _gate_kernel,
        grid=(nblk,),
        in_specs=[
            pl.BlockSpec((T_BLK, D_MODEL), lambda i: (i, 0)),
            pl.BlockSpec((NUM_PATHS, D_MODEL), lambda i: (0, 0)),
        ],
        out_specs=[
            pl.BlockSpec((V, T_BLK), lambda i: (0, i)),
            pl.BlockSpec((V, T_BLK), lambda i: (0, i)),
        ],
        out_shape=[
            jax.ShapeDtypeStruct((V, n_tokens), jnp.int32),
            jax.ShapeDtypeStruct((V, n_tokens), jnp.float32),
        ],
    )(x, W)
    return idx_t[:K].T, scores_t[:K].T


# speedup vs baseline: 1.2723x; 1.2723x over previous
import jax
import jax.numpy as jnp
from jax.experimental import pallas as pl
from jax.experimental.pallas import tpu as pltpu

D_MODEL = 4096
NUM_PATHS = 512
K = 51
T_BLK = 512
SPLIT = 2
V = 64


def _cex(ks, ix, i, j, flip):
    a, b = ks[i], ks[j]
    ia, ib = ix[i], ix[j]
    g = a < b
    if not flip:
        ks[i], ks[j] = jnp.where(g, b, a), jnp.where(g, a, b)
        ix[i], ix[j] = jnp.where(g, ib, ia), jnp.where(g, ia, ib)
    else:
        ks[i], ks[j] = jnp.where(g, a, b), jnp.where(g, b, a)
        ix[i], ix[j] = jnp.where(g, ia, ib), jnp.where(g, ib, ia)


def _sort64_desc(ks, ix):
    def merge(lo, hi, r):
        step = r * 2
        if step < hi - lo:
            merge(lo, hi, step)
            merge(lo + r, hi, step)
            for i in range(lo + r, hi - r, step):
                _cex(ks, ix, i, i + r, flip=False)
        else:
            _cex(ks, ix, lo, lo + r, flip=False)

    def msort(lo, hi):
        if hi - lo > 1:
            mid = (lo + hi) // 2
            msort(lo, mid)
            msort(mid, hi)
            merge(lo, hi, 1)

    msort(0, V)


def _merge64_desc(ks, ix):
    s = V // 2
    while s >= 1:
        for i in range(V):
            if not i & s:
                _cex(ks, ix, i, i | s, flip=False)
        s //= 2


def _subrot(arr, d):
    return pltpu.roll(arr, 8 - d, axis=0)


def _gate_kernel(x_ref, w_ref, idx_ref, scores_ref):
    w = w_ref[...]
    ts = T_BLK // SPLIT
    logits_parts = []
    for p in range(SPLIT):
        xp = x_ref[p * ts:(p + 1) * ts, :]
        logits_parts.append(jax.lax.dot_general(
            w, xp, (((1,), (1,)), ((), ())),
            preferred_element_type=jnp.float32,
        ))
    for p in range(SPLIT):
        _topk_softmax(logits_parts[p], idx_ref, scores_ref, p * ts)


def _topk_softmax(logits, idx_ref, scores_ref, col0):
    t = logits.shape[-1]
    ks = [logits[8 * v: 8 * v + 8, :] for v in range(V)]
    sub = jax.lax.broadcasted_iota(jnp.int32, (8, t), 0)
    ix = [sub + 8 * v for v in range(V)]

    _sort64_desc(ks, ix)

    for d in (1, 2, 4):
        pks = [_subrot(ks[V - 1 - v], d) for v in range(V)]
        pix = [_subrot(ix[V - 1 - v], d) for v in range(V)]
        for v in range(V):
            g = pks[v] > ks[v]
            ks[v] = jnp.where(g, pks[v], ks[v])
            ix[v] = jnp.where(g, pix[v], ix[v])
        _merge64_desc(ks, ix)

    kv = jnp.concatenate([ks[r][0:1, :] for r in range(V)], axis=0)
    iv = jnp.concatenate([ix[r][0:1, :] for r in range(V)], axis=0)

    rank = jax.lax.broadcasted_iota(jnp.int32, (V, t), 0)
    e = jnp.where(rank < K, jnp.exp(kv - kv[0:1, :]), 0.0)
    ssum = jnp.sum(e, axis=0, keepdims=True)
    sc = e / ssum

    idx_ref[:, col0:col0 + t] = iv
    scores_ref[:, col0:col0 + t] = sc


@jax.jit
def kernel(x, W):
    n_tokens = x.shape[0]
    nblk = n_tokens // T_BLK
    idx_t, scores_t = pl.pallas_call(
        _gate_kernel,
        grid=(nblk,),
        in_specs=[
            pl.BlockSpec((T_BLK, D_MODEL), lambda i: (i, 0)),
            pl.BlockSpec((NUM_PATHS, D_MODEL), lambda i: (0, 0)),
        ],
        out_specs=[
            pl.BlockSpec((V, T_BLK), lambda i: (0, i)),
            pl.BlockSpec((V, T_BLK), lambda i: (0, i)),
        ],
        out_shape=[
            jax.ShapeDtypeStruct((V, n_tokens), jnp.int32),
            jax.ShapeDtypeStruct((V, n_tokens), jnp.float32),
        ],
    )(x, W)
    return idx_t[:K].T, scores_t[:K].T


# speedup vs baseline: 1.2743x; 1.0016x over previous
import jax
import jax.numpy as jnp
from jax.experimental import pallas as pl
from jax.experimental.pallas import tpu as pltpu

D_MODEL = 4096
NUM_PATHS = 512
K = 51
T_BLK = 512
SPLIT = 2
SORT_SPLIT = 4
V = 64


def _cex(ks, ix, i, j, flip):
    a, b = ks[i], ks[j]
    ia, ib = ix[i], ix[j]
    g = a < b
    if not flip:
        ks[i], ks[j] = jnp.where(g, b, a), jnp.where(g, a, b)
        ix[i], ix[j] = jnp.where(g, ib, ia), jnp.where(g, ia, ib)
    else:
        ks[i], ks[j] = jnp.where(g, a, b), jnp.where(g, b, a)
        ix[i], ix[j] = jnp.where(g, ia, ib), jnp.where(g, ib, ia)


def _sort64_desc(ks, ix):
    def merge(lo, hi, r):
        step = r * 2
        if step < hi - lo:
            merge(lo, hi, step)
            merge(lo + r, hi, step)
            for i in range(lo + r, hi - r, step):
                _cex(ks, ix, i, i + r, flip=False)
        else:
            _cex(ks, ix, lo, lo + r, flip=False)

    def msort(lo, hi):
        if hi - lo > 1:
            mid = (lo + hi) // 2
            msort(lo, mid)
            msort(mid, hi)
            merge(lo, hi, 1)

    msort(0, V)


def _merge64_desc(ks, ix):
    s = V // 2
    while s >= 1:
        for i in range(V):
            if not i & s:
                _cex(ks, ix, i, i | s, flip=False)
        s //= 2


def _subrot(arr, d):
    return pltpu.roll(arr, 8 - d, axis=0)


def _gate_kernel(x_ref, w_ref, idx_ref, scores_ref):
    w = w_ref[...]
    ts = T_BLK // SPLIT
    logits_parts = []
    for p in range(SPLIT):
        xp = x_ref[p * ts:(p + 1) * ts, :]
        logits_parts.append(jax.lax.dot_general(
            w, xp, (((1,), (1,)), ((), ())),
            preferred_element_type=jnp.float32,
        ))
    tq = T_BLK // SORT_SPLIT
    for q in range(SORT_SPLIT):
        part = logits_parts[(q * tq) // ts]
        off = (q * tq) % ts
        _topk_softmax(part[:, off:off + tq], idx_ref, scores_ref, q * tq)


def _topk_softmax(logits, idx_ref, scores_ref, col0):
    t = logits.shape[-1]
    ks = [logits[8 * v: 8 * v + 8, :] for v in range(V)]
    sub = jax.lax.broadcasted_iota(jnp.int32, (8, t), 0)
    ix = [sub + 8 * v for v in range(V)]

    _sort64_desc(ks, ix)

    for d in (1, 2, 4):
        pks = [_subrot(ks[V - 1 - v], d) for v in range(V)]
        pix = [_subrot(ix[V - 1 - v], d) for v in range(V)]
        for v in range(V):
            g = pks[v] > ks[v]
            ks[v] = jnp.where(g, pks[v], ks[v])
            ix[v] = jnp.where(g, pix[v], ix[v])
        _merge64_desc(ks, ix)

    kv = jnp.concatenate([ks[r][0:1, :] for r in range(V)], axis=0)
    iv = jnp.concatenate([ix[r][0:1, :] for r in range(V)], axis=0)

    rank = jax.lax.broadcasted_iota(jnp.int32, (V, t), 0)
    e = jnp.where(rank < K, jnp.exp(kv - kv[0:1, :]), 0.0)
    ssum = jnp.sum(e, axis=0, keepdims=True)
    sc = e / ssum

    idx_ref[:, col0:col0 + t] = iv
    scores_ref[:, col0:col0 + t] = sc


@jax.jit
def kernel(x, W):
    n_tokens = x.shape[0]
    nblk = n_tokens // T_BLK
    idx_t, scores_t = pl.pallas_call(
        _gate_kernel,
        grid=(nblk,),
        in_specs=[
            pl.BlockSpec((T_BLK, D_MODEL), lambda i: (i, 0)),
            pl.BlockSpec((NUM_PATHS, D_MODEL), lambda i: (0, 0)),
        ],
        out_specs=[
            pl.BlockSpec((V, T_BLK), lambda i: (0, i)),
            pl.BlockSpec((V, T_BLK), lambda i: (0, i)),
        ],
        out_shape=[
            jax.ShapeDtypeStruct((V, n_tokens), jnp.int32),
            jax.ShapeDtypeStruct((V, n_tokens), jnp.float32),
        ],
    )(x, W)
    return idx_t[:K].T, scores_t[:K].T


# speedup vs baseline: 1.5149x; 1.1888x over previous
import jax
import jax.numpy as jnp
from jax.experimental import pallas as pl
from jax.experimental.pallas import tpu as pltpu

D_MODEL = 4096
NUM_PATHS = 512
K = 51
T_BLK = 512
SPLIT = 2
SORT_SPLIT = 4
V = 64


def _cex(ks, ix, i, j, flip):
    a, b = ks[i], ks[j]
    ia, ib = ix[i], ix[j]
    g = a < b
    if not flip:
        ks[i], ks[j] = jnp.where(g, b, a), jnp.where(g, a, b)
        ix[i], ix[j] = jnp.where(g, ib, ia), jnp.where(g, ia, ib)
    else:
        ks[i], ks[j] = jnp.where(g, a, b), jnp.where(g, b, a)
        ix[i], ix[j] = jnp.where(g, ia, ib), jnp.where(g, ib, ia)


def _sort64_desc(ks, ix):
    def merge(lo, hi, r):
        step = r * 2
        if step < hi - lo:
            merge(lo, hi, step)
            merge(lo + r, hi, step)
            for i in range(lo + r, hi - r, step):
                _cex(ks, ix, i, i + r, flip=False)
        else:
            _cex(ks, ix, lo, lo + r, flip=False)

    def msort(lo, hi):
        if hi - lo > 1:
            mid = (lo + hi) // 2
            msort(lo, mid)
            msort(mid, hi)
            merge(lo, hi, 1)

    msort(0, V)


def _merge64_desc(ks, ix):
    s = V // 2
    while s >= 1:
        for i in range(V):
            if not i & s:
                _cex(ks, ix, i, i | s, flip=False)
        s //= 2


def _subrot(arr, d):
    return pltpu.roll(arr, 8 - d, axis=0)


def _gate_kernel(x_ref, w_ref, idx_ref, scores_ref):
    w = w_ref[...]
    ts = T_BLK // SPLIT
    logits_parts = []
    for p in range(SPLIT):
        xp = x_ref[p * ts:(p + 1) * ts, :]
        logits_parts.append(jax.lax.dot_general(
            w, xp, (((1,), (1,)), ((), ())),
            preferred_element_type=jnp.float32,
        ))
    tq = T_BLK // SORT_SPLIT
    for q in range(0, SORT_SPLIT, 2):
        parts = []
        for qq in (q, q + 1):
            part = logits_parts[(qq * tq) // ts]
            off = (qq * tq) % ts
            parts.append(part[:, off:off + tq])
        _topk_pair(parts[0], parts[1], idx_ref, scores_ref, q * tq, tq)


def _phase_a_combine(logits, rot):
    t = logits.shape[-1]
    ks = [logits[8 * v: 8 * v + 8, :] for v in range(V)]
    sub = jax.lax.broadcasted_iota(jnp.int32, (8, t), 0)
    ix = [sub + 8 * v for v in range(V)]
    _sort64_desc(ks, ix)
    pks = [pltpu.roll(ks[V - 1 - v], rot, axis=0) for v in range(V)]
    pix = [pltpu.roll(ix[V - 1 - v], rot, axis=0) for v in range(V)]
    for v in range(V):
        g = pks[v] > ks[v]
        ks[v] = jnp.where(g, pks[v], ks[v])
        ix[v] = jnp.where(g, pix[v], ix[v])
    return ks, ix


def _topk_pair(lg_a, lg_b, idx_ref, scores_ref, col0, tq):
    ks_a, ix_a = _phase_a_combine(lg_a, 1)
    ks_b, ix_b = _phase_a_combine(lg_b, 7)
    even = (jax.lax.broadcasted_iota(
        jnp.int32, (8, tq), 0) % 2) == 0
    ks = [jnp.where(even, ks_a[v], ks_b[v]) for v in range(V)]
    ix = [jnp.where(even, ix_a[v], ix_b[v]) for v in range(V)]
    _merge64_desc(ks, ix)
    for d in (2, 4):
        pks = [_subrot(ks[V - 1 - v], d) for v in range(V)]
        pix = [_subrot(ix[V - 1 - v], d) for v in range(V)]
        for v in range(V):
            g = pks[v] > ks[v]
            ks[v] = jnp.where(g, pks[v], ks[v])
            ix[v] = jnp.where(g, pix[v], ix[v])
        _merge64_desc(ks, ix)
    _emit(ks, ix, 0, idx_ref, scores_ref, col0, tq)
    _emit(ks, ix, 1, idx_ref, scores_ref, col0 + tq, tq)


def _emit(ks, ix, sl, idx_ref, scores_ref, col0, tq):
    kv = jnp.concatenate([ks[r][sl:sl + 1, :] for r in range(V)], axis=0)
    iv = jnp.concatenate([ix[r][sl:sl + 1, :] for r in range(V)], axis=0)

    rank = jax.lax.broadcasted_iota(jnp.int32, (V, tq), 0)
    e = jnp.where(rank < K, jnp.exp(kv - kv[0:1, :]), 0.0)
    ssum = jnp.sum(e, axis=0, keepdims=True)
    sc = e / ssum

    idx_ref[:, col0:col0 + tq] = iv
    scores_ref[:, col0:col0 + tq] = sc


@jax.jit
def kernel(x, W):
    n_tokens = x.shape[0]
    nblk = n_tokens // T_BLK
    idx_t, scores_t = pl.pallas_call(
        _gate_kernel,
        grid=(nblk,),
        in_specs=[
            pl.BlockSpec((T_BLK, D_MODEL), lambda i: (i, 0)),
            pl.BlockSpec((NUM_PATHS, D_MODEL), lambda i: (0, 0)),
        ],
        out_specs=[
            pl.BlockSpec((V, T_BLK), lambda i: (0, i)),
            pl.BlockSpec((V, T_BLK), lambda i: (0, i)),
        ],
        out_shape=[
            jax.ShapeDtypeStruct((V, n_tokens), jnp.int32),
            jax.ShapeDtypeStruct((V, n_tokens), jnp.float32),
        ],
    )(x, W)
    return idx_t[:K].T, scores_t[:K].T


# speedup vs baseline: 1.5247x; 1.0065x over previous
import jax
import jax.numpy as jnp
from jax.experimental import pallas as pl
from jax.experimental.pallas import tpu as pltpu

D_MODEL = 4096
NUM_PATHS = 512
K = 51
T_BLK = 1024
SPLIT = 2
SORT_SPLIT = 8
V = 64


def _cex(ks, ix, i, j, flip):
    a, b = ks[i], ks[j]
    ia, ib = ix[i], ix[j]
    g = a < b
    if not flip:
        ks[i], ks[j] = jnp.where(g, b, a), jnp.where(g, a, b)
        ix[i], ix[j] = jnp.where(g, ib, ia), jnp.where(g, ia, ib)
    else:
        ks[i], ks[j] = jnp.where(g, a, b), jnp.where(g, b, a)
        ix[i], ix[j] = jnp.where(g, ia, ib), jnp.where(g, ib, ia)


def _sort64_desc(ks, ix):
    def merge(lo, hi, r):
        step = r * 2
        if step < hi - lo:
            merge(lo, hi, step)
            merge(lo + r, hi, step)
            for i in range(lo + r, hi - r, step):
                _cex(ks, ix, i, i + r, flip=False)
        else:
            _cex(ks, ix, lo, lo + r, flip=False)

    def msort(lo, hi):
        if hi - lo > 1:
            mid = (lo + hi) // 2
            msort(lo, mid)
            msort(mid, hi)
            merge(lo, hi, 1)

    msort(0, V)


def _merge64_desc(ks, ix):
    s = V // 2
    while s >= 1:
        for i in range(V):
            if not i & s:
                _cex(ks, ix, i, i | s, flip=False)
        s //= 2


def _subrot(arr, d):
    return pltpu.roll(arr, 8 - d, axis=0)


def _gate_kernel(x_ref, w_ref, idx_ref, scores_ref):
    w = w_ref[...]
    ts = T_BLK // SPLIT
    logits_parts = []
    for p in range(SPLIT):
        xp = x_ref[p * ts:(p + 1) * ts, :]
        logits_parts.append(jax.lax.dot_general(
            w, xp, (((1,), (1,)), ((), ())),
            preferred_element_type=jnp.float32,
        ))
    tq = T_BLK // SORT_SPLIT
    for q in range(0, SORT_SPLIT, 2):
        parts = []
        for qq in (q, q + 1):
            part = logits_parts[(qq * tq) // ts]
            off = (qq * tq) % ts
            parts.append(part[:, off:off + tq])
        _topk_pair(parts[0], parts[1], idx_ref, scores_ref, q * tq, tq)


def _phase_a_combine(logits, rot):
    t = logits.shape[-1]
    ks = [logits[8 * v: 8 * v + 8, :] for v in range(V)]
    sub = jax.lax.broadcasted_iota(jnp.int32, (8, t), 0)
    ix = [sub + 8 * v for v in range(V)]
    _sort64_desc(ks, ix)
    pks = [pltpu.roll(ks[V - 1 - v], rot, axis=0) for v in range(V)]
    pix = [pltpu.roll(ix[V - 1 - v], rot, axis=0) for v in range(V)]
    for v in range(V):
        g = pks[v] > ks[v]
        ks[v] = jnp.where(g, pks[v], ks[v])
        ix[v] = jnp.where(g, pix[v], ix[v])
    return ks, ix


def _topk_pair(lg_a, lg_b, idx_ref, scores_ref, col0, tq):
    ks_a, ix_a = _phase_a_combine(lg_a, 1)
    ks_b, ix_b = _phase_a_combine(lg_b, 7)
    even = (jax.lax.broadcasted_iota(
        jnp.int32, (8, tq), 0) % 2) == 0
    ks = [jnp.where(even, ks_a[v], ks_b[v]) for v in range(V)]
    ix = [jnp.where(even, ix_a[v], ix_b[v]) for v in range(V)]
    _merge64_desc(ks, ix)
    for d in (2, 4):
        pks = [_subrot(ks[V - 1 - v], d) for v in range(V)]
        pix = [_subrot(ix[V - 1 - v], d) for v in range(V)]
        for v in range(V):
            g = pks[v] > ks[v]
            ks[v] = jnp.where(g, pks[v], ks[v])
            ix[v] = jnp.where(g, pix[v], ix[v])
        _merge64_desc(ks, ix)
    _emit(ks, ix, 0, idx_ref, scores_ref, col0, tq)
    _emit(ks, ix, 1, idx_ref, scores_ref, col0 + tq, tq)


def _emit(ks, ix, sl, idx_ref, scores_ref, col0, tq):
    kv = jnp.concatenate([ks[r][sl:sl + 1, :] for r in range(V)], axis=0)
    iv = jnp.concatenate([ix[r][sl:sl + 1, :] for r in range(V)], axis=0)

    rank = jax.lax.broadcasted_iota(jnp.int32, (V, tq), 0)
    e = jnp.where(rank < K, jnp.exp(kv - kv[0:1, :]), 0.0)
    ssum = jnp.sum(e, axis=0, keepdims=True)
    sc = e / ssum

    idx_ref[:, col0:col0 + tq] = iv
    scores_ref[:, col0:col0 + tq] = sc


@jax.jit
def kernel(x, W):
    n_tokens = x.shape[0]
    nblk = n_tokens // T_BLK
    idx_t, scores_t = pl.pallas_call(
        _gate_kernel,
        grid=(nblk,),
        in_specs=[
            pl.BlockSpec((T_BLK, D_MODEL), lambda i: (i, 0)),
            pl.BlockSpec((NUM_PATHS, D_MODEL), lambda i: (0, 0)),
        ],
        out_specs=[
            pl.BlockSpec((V, T_BLK), lambda i: (0, i)),
            pl.BlockSpec((V, T_BLK), lambda i: (0, i)),
        ],
        out_shape=[
            jax.ShapeDtypeStruct((V, n_tokens), jnp.int32),
            jax.ShapeDtypeStruct((V, n_tokens), jnp.float32),
        ],
    )(x, W)
    return idx_t[:K].T, scores_t[:K].T
